# baseline (device time: 69201 ns/iter reference)
import jax
import jax.numpy as jnp
from jax import lax
from jax.experimental import pallas as pl
from jax.experimental.pallas import tpu as pltpu

N_DEV = 4
SQ = 1024
SKV = 1024
H_PER = 8
DH = 128
D_MODEL = 1024
BLK = 64
QB = 256
NQB = SQ // QB
CR = 64
NC = SQ // CR
CPB = QB // CR
W_OWN = NC // N_DEV
SCALE = 0.08838834764831843


def kernel(x, Wq, K_ext, V_ext, Wo):
    pos = lax.axis_index("i")
    x2 = x[0]
    Wq_sl = lax.dynamic_slice_in_dim(Wq, pos * D_MODEL, D_MODEL, axis=1)
    Wo_sl = lax.dynamic_slice_in_dim(Wo, pos * D_MODEL, D_MODEL, axis=0)
    K2 = K_ext.reshape(SKV, H_PER * DH)
    V2 = V_ext.reshape(SKV, H_PER * DH)

    def body(x_ref, wq_ref, k_ref, v_ref, wo_ref, out_ref,
             ctx_ref, stage_ref, rs_ref, s1_sems, r1_sems, s2_sems, r2_sems):
        m = lax.axis_index("i")

        barrier_sem = pltpu.get_barrier_semaphore()
        for d in range(1, N_DEV):
            pl.semaphore_signal(
                barrier_sem, inc=1,
                device_id=(lax.rem(m + d, N_DEV),),
                device_id_type=pl.DeviceIdType.MESH,
            )
        pl.semaphore_wait(barrier_sem, N_DEV - 1)

        for c in range(NQB):
            rows = slice(c * QB, (c + 1) * QB)
            kvlen = QB * (c + 1)
            q_blk = jnp.dot(x_ref[rows, :], wq_ref[...],
                            preferred_element_type=jnp.float32)

            row_blk = (lax.broadcasted_iota(jnp.int32, (QB, kvlen), 0)
                       + c * QB) // BLK
            col_blk = lax.broadcasted_iota(jnp.int32, (QB, kvlen), 1) // BLK
            mask = col_blk <= row_blk

            for h in range(H_PER):
                sl = slice(h * DH, (h + 1) * DH)
                s = lax.dot_general(
                    q_blk[:, sl], k_ref[:kvlen, sl],
                    (((1,), (1,)), ((), ())),
                    preferred_element_type=jnp.float32) * SCALE
                s = jnp.where(mask, s, -1e9)
                mx = jnp.max(s, axis=1, keepdims=True)
                w = jnp.exp(s - mx)
                p = w / jnp.sum(w, axis=1, keepdims=True)
                ctx_ref[rows, sl] = jnp.dot(
                    p, v_ref[:kvlen, sl], preferred_element_type=jnp.float32)

            for t in range(CPB):
                j = c * CPB + t
                w_idx = j // N_DEV
                o = j % N_DEV
                pr = jnp.dot(ctx_ref[j * CR:(j + 1) * CR, :], wo_ref[...],
                             preferred_element_type=jnp.float32)
                stage_ref[j] = pr

                @pl.when(m == o)
                def _(w_idx=w_idx, pr=pr):
                    rs_ref[w_idx, m] = pr

                @pl.when(m != o)
                def _(j=j, w_idx=w_idx, o=o):
                    rdma = pltpu.make_async_remote_copy(
                        src_ref=stage_ref.at[j],
                        dst_ref=rs_ref.at[w_idx, m],
                        send_sem=s1_sems.at[j],
                        recv_sem=r1_sems.at[w_idx, m],
                        device_id=(o,),
                        device_id_type=pl.DeviceIdType.MESH,
                    )
                    rdma.start()

        for w_idx in range(W_OWN):
            for s_ in range(N_DEV):
                @pl.when(s_ != m)
                def _(w_idx=w_idx, s_=s_):
                    rd = pltpu.make_async_remote_copy(
                        src_ref=rs_ref.at[w_idx, s_],
                        dst_ref=rs_ref.at[w_idx, s_],
                        send_sem=s1_sems.at[0],
                        recv_sem=r1_sems.at[w_idx, s_],
                        device_id=(0,),
                        device_id_type=pl.DeviceIdType.MESH,
                    )
                    rd.wait_recv()
            total = (rs_ref[w_idx, 0] + rs_ref[w_idx, 1]
                     + rs_ref[w_idx, 2] + rs_ref[w_idx, 3])
            roff = (w_idx * N_DEV + m) * CR
            out_ref[pl.ds(roff, CR), :] = total
            for d in range(1, N_DEV):
                tgt = lax.rem(m + d, N_DEV)
                rdma = pltpu.make_async_remote_copy(
                    src_ref=out_ref.at[pl.ds(roff, CR)],
                    dst_ref=out_ref.at[pl.ds(roff, CR)],
                    send_sem=s2_sems.at[w_idx, d - 1],
                    recv_sem=r2_sems.at[m, w_idx],
                    device_id=(tgt,),
                    device_id_type=pl.DeviceIdType.MESH,
                )
                rdma.start()

        for s_ in range(N_DEV):
            for w_idx in range(W_OWN):
                @pl.when(s_ != m)
                def _(s_=s_, w_idx=w_idx):
                    rd = pltpu.make_async_remote_copy(
                        src_ref=out_ref.at[pl.ds((w_idx * N_DEV + s_) * CR,
                                                 CR)],
                        dst_ref=out_ref.at[pl.ds((w_idx * N_DEV + s_) * CR,
                                                 CR)],
                        send_sem=s2_sems.at[0, 0],
                        recv_sem=r2_sems.at[s_, w_idx],
                        device_id=(0,),
                        device_id_type=pl.DeviceIdType.MESH,
                    )
                    rd.wait_recv()

        for j in range(NC):
            @pl.when(m != j % N_DEV)
            def _(j=j):
                rd = pltpu.make_async_remote_copy(
                    src_ref=stage_ref.at[j],
                    dst_ref=rs_ref.at[j // N_DEV, m],
                    send_sem=s1_sems.at[j],
                    recv_sem=r1_sems.at[0, 0],
                    device_id=(0,),
                    device_id_type=pl.DeviceIdType.MESH,
                )
                rd.wait_send()
        for w_idx in range(W_OWN):
            for d in range(1, N_DEV):
                rd = pltpu.make_async_remote_copy(
                    src_ref=out_ref.at[pl.ds(0, CR)],
                    dst_ref=out_ref.at[pl.ds(0, CR)],
                    send_sem=s2_sems.at[w_idx, d - 1],
                    recv_sem=r2_sems.at[0, 0],
                    device_id=(0,),
                    device_id_type=pl.DeviceIdType.MESH,
                )
                rd.wait_send()

    out = pl.pallas_call(
        body,
        out_shape=jax.ShapeDtypeStruct((SQ, D_MODEL), jnp.float32),
        in_specs=[pl.BlockSpec(memory_space=pltpu.VMEM)] * 5,
        out_specs=pl.BlockSpec(memory_space=pltpu.VMEM),
        scratch_shapes=[
            pltpu.VMEM((SQ, H_PER * DH), jnp.float32),
            pltpu.VMEM((NC, CR, D_MODEL), jnp.float32),
            pltpu.VMEM((W_OWN, N_DEV, CR, D_MODEL), jnp.float32),
            pltpu.SemaphoreType.DMA((NC,)),
            pltpu.SemaphoreType.DMA((W_OWN, N_DEV)),
            pltpu.SemaphoreType.DMA((W_OWN, N_DEV - 1)),
            pltpu.SemaphoreType.DMA((N_DEV, W_OWN)),
        ],
        compiler_params=pltpu.CompilerParams(collective_id=0),
    )(x2, Wq_sl, K2, V2, Wo_sl)

    return out.reshape(1, SQ, D_MODEL)


# device time: 48250 ns/iter; 1.4342x vs baseline; 1.4342x over previous
import jax
import jax.numpy as jnp
from jax import lax
from jax.experimental import pallas as pl
from jax.experimental.pallas import tpu as pltpu

N_DEV = 4
SQ = 1024
SKV = 1024
H_PER = 8
DH = 128
D_MODEL = 1024
BLK = 64
QB = 256
NQB = SQ // QB
CR = 64
NC = SQ // CR
CPB = QB // CR
W_OWN = NC // N_DEV
SCALE = 0.08838834764831843
COMM_DT = jnp.bfloat16


def kernel(x, Wq, K_ext, V_ext, Wo):
    x2 = x[0]
    K2 = K_ext.reshape(SKV, H_PER * DH)
    V2 = V_ext.reshape(SKV, H_PER * DH)

    def body(x_ref, wq_ref, k_ref, v_ref, wo_ref, out_ref,
             wq_sl, wo_sl, ctx_ref, stage_ref, rs_ref, ag_ref,
             w_sems, s1_sems, r1_sems, s2_sems, r2_sems):
        m = lax.axis_index("i")

        wq_dma = pltpu.make_async_copy(
            wq_ref.at[:, pl.ds(m * D_MODEL, D_MODEL)], wq_sl, w_sems.at[0])
        wq_dma.start()
        wo_dma = pltpu.make_async_copy(
            wo_ref.at[pl.ds(m * D_MODEL, D_MODEL), :], wo_sl, w_sems.at[1])
        wo_dma.start()

        barrier_sem = pltpu.get_barrier_semaphore()
        for d in range(1, N_DEV):
            pl.semaphore_signal(
                barrier_sem, inc=1,
                device_id=(lax.rem(m + d, N_DEV),),
                device_id_type=pl.DeviceIdType.MESH,
            )
        pl.semaphore_wait(barrier_sem, N_DEV - 1)
        wq_dma.wait()
        wo_dma.wait()

        for c in range(NQB):
            rows = slice(c * QB, (c + 1) * QB)
            kvlen = QB * (c + 1)
            q_blk = jnp.dot(x_ref[rows, :], wq_sl[...],
                            preferred_element_type=jnp.float32)

            row_blk = (lax.broadcasted_iota(jnp.int32, (QB, kvlen), 0)
                       + c * QB) // BLK
            col_blk = lax.broadcasted_iota(jnp.int32, (QB, kvlen), 1) // BLK
            mask = col_blk <= row_blk

            for h in range(H_PER):
                sl = slice(h * DH, (h + 1) * DH)
                s = lax.dot_general(
                    q_blk[:, sl], k_ref[:kvlen, sl],
                    (((1,), (1,)), ((), ())),
                    preferred_element_type=jnp.float32) * SCALE
                s = jnp.where(mask, s, -1e9)
                mx = jnp.max(s, axis=1, keepdims=True)
                w = jnp.exp(s - mx)
                p = w / jnp.sum(w, axis=1, keepdims=True)
                ctx_ref[rows, sl] = jnp.dot(
                    p, v_ref[:kvlen, sl], preferred_element_type=jnp.float32)

            for t in range(CPB):
                j = c * CPB + t
                w_idx = j // N_DEV
                o = j % N_DEV
                pr = jnp.dot(ctx_ref[j * CR:(j + 1) * CR, :], wo_sl[...],
                             preferred_element_type=jnp.float32)
                pr16 = pr.astype(COMM_DT)
                stage_ref[j] = pr16

                @pl.when(m == o)
                def _(w_idx=w_idx, pr16=pr16):
                    rs_ref[w_idx, m] = pr16

                @pl.when(m != o)
                def _(j=j, w_idx=w_idx, o=o):
                    rdma = pltpu.make_async_remote_copy(
                        src_ref=stage_ref.at[j],
                        dst_ref=rs_ref.at[w_idx, m],
                        send_sem=s1_sems.at[j],
                        recv_sem=r1_sems.at[w_idx, m],
                        device_id=(o,),
                        device_id_type=pl.DeviceIdType.MESH,
                    )
                    rdma.start()

        for w_idx in range(W_OWN):
            for s_ in range(N_DEV):
                @pl.when(s_ != m)
                def _(w_idx=w_idx, s_=s_):
                    rd = pltpu.make_async_remote_copy(
                        src_ref=rs_ref.at[w_idx, s_],
                        dst_ref=rs_ref.at[w_idx, s_],
                        send_sem=s1_sems.at[0],
                        recv_sem=r1_sems.at[w_idx, s_],
                        device_id=(0,),
                        device_id_type=pl.DeviceIdType.MESH,
                    )
                    rd.wait_recv()
            total = (rs_ref[w_idx, 0].astype(jnp.float32)
                     + rs_ref[w_idx, 1].astype(jnp.float32)
                     + rs_ref[w_idx, 2].astype(jnp.float32)
                     + rs_ref[w_idx, 3].astype(jnp.float32))
            j_own = w_idx * N_DEV + m
            out_ref[pl.ds(j_own * CR, CR), :] = total
            ag_ref[j_own] = total.astype(COMM_DT)
            for d in range(1, N_DEV):
                tgt = lax.rem(m + d, N_DEV)
                rdma = pltpu.make_async_remote_copy(
                    src_ref=ag_ref.at[j_own],
                    dst_ref=ag_ref.at[j_own],
                    send_sem=s2_sems.at[w_idx, d - 1],
                    recv_sem=r2_sems.at[m, w_idx],
                    device_id=(tgt,),
                    device_id_type=pl.DeviceIdType.MESH,
                )
                rdma.start()

        for s_ in range(N_DEV):
            for w_idx in range(W_OWN):
                @pl.when(s_ != m)
                def _(s_=s_, w_idx=w_idx):
                    jj = w_idx * N_DEV + s_
                    rd = pltpu.make_async_remote_copy(
                        src_ref=ag_ref.at[jj],
                        dst_ref=ag_ref.at[jj],
                        send_sem=s2_sems.at[0, 0],
                        recv_sem=r2_sems.at[s_, w_idx],
                        device_id=(0,),
                        device_id_type=pl.DeviceIdType.MESH,
                    )
                    rd.wait_recv()
                    out_ref[jj * CR:(jj + 1) * CR, :] = (
                        ag_ref[jj].astype(jnp.float32))

        for j in range(NC):
            @pl.when(m != j % N_DEV)
            def _(j=j):
                rd = pltpu.make_async_remote_copy(
                    src_ref=stage_ref.at[j],
                    dst_ref=rs_ref.at[j // N_DEV, m],
                    send_sem=s1_sems.at[j],
                    recv_sem=r1_sems.at[0, 0],
                    device_id=(0,),
                    device_id_type=pl.DeviceIdType.MESH,
                )
                rd.wait_send()
        for w_idx in range(W_OWN):
            for d in range(1, N_DEV):
                rd = pltpu.make_async_remote_copy(
                    src_ref=ag_ref.at[0],
                    dst_ref=ag_ref.at[0],
                    send_sem=s2_sems.at[w_idx, d - 1],
                    recv_sem=r2_sems.at[0, 0],
                    device_id=(0,),
                    device_id_type=pl.DeviceIdType.MESH,
                )
                rd.wait_send()

    out = pl.pallas_call(
        body,
        out_shape=jax.ShapeDtypeStruct((SQ, D_MODEL), jnp.float32),
        in_specs=[
            pl.BlockSpec(memory_space=pltpu.VMEM),
            pl.BlockSpec(memory_space=pltpu.MemorySpace.HBM),
            pl.BlockSpec(memory_space=pltpu.VMEM),
            pl.BlockSpec(memory_space=pltpu.VMEM),
            pl.BlockSpec(memory_space=pltpu.MemorySpace.HBM),
        ],
        out_specs=pl.BlockSpec(memory_space=pltpu.VMEM),
        scratch_shapes=[
            pltpu.VMEM((D_MODEL, D_MODEL), jnp.float32),
            pltpu.VMEM((D_MODEL, D_MODEL), jnp.float32),
            pltpu.VMEM((SQ, H_PER * DH), jnp.float32),
            pltpu.VMEM((NC, CR, D_MODEL), COMM_DT),
            pltpu.VMEM((W_OWN, N_DEV, CR, D_MODEL), COMM_DT),
            pltpu.VMEM((NC, CR, D_MODEL), COMM_DT),
            pltpu.SemaphoreType.DMA((2,)),
            pltpu.SemaphoreType.DMA((NC,)),
            pltpu.SemaphoreType.DMA((W_OWN, N_DEV)),
            pltpu.SemaphoreType.DMA((W_OWN, N_DEV - 1)),
            pltpu.SemaphoreType.DMA((N_DEV, W_OWN)),
        ],
        compiler_params=pltpu.CompilerParams(collective_id=0),
    )(x2, Wq, K2, V2, Wo)

    return out.reshape(1, SQ, D_MODEL)


# device time: 42678 ns/iter; 1.6215x vs baseline; 1.1306x over previous
import jax
import jax.numpy as jnp
from jax import lax
from jax.experimental import pallas as pl
from jax.experimental.pallas import tpu as pltpu

N_DEV = 4
SQ = 1024
SKV = 1024
H_PER = 8
DH = 128
D_MODEL = 1024
BLK = 64
QB = 256
NQB = SQ // QB
CR = 64
NC = SQ // CR
CPB = QB // CR
W_OWN = NC // N_DEV
SCALE = 0.08838834764831843
COMM_DT = jnp.bfloat16


def kernel(x, Wq, K_ext, V_ext, Wo):
    x2 = x[0]
    K2 = K_ext.reshape(SKV, H_PER * DH)
    V2 = V_ext.reshape(SKV, H_PER * DH)

    def body(x_ref, wq_ref, k_ref, v_ref, wo_ref, out_ref,
             wq_sl, wo_sl, ctx_ref, stage_ref, rs_ref, ag_ref,
             w_sems, s1_sems, r1_sems, s2_sems, r2_sems):
        m = lax.axis_index("i")

        wq_dma = pltpu.make_async_copy(
            wq_ref.at[:, pl.ds(m * D_MODEL, D_MODEL)], wq_sl, w_sems.at[0])
        wq_dma.start()
        wo_dma = pltpu.make_async_copy(
            wo_ref.at[pl.ds(m * D_MODEL, D_MODEL), :], wo_sl, w_sems.at[1])
        wo_dma.start()

        barrier_sem = pltpu.get_barrier_semaphore()
        for d in range(1, N_DEV):
            pl.semaphore_signal(
                barrier_sem, inc=1,
                device_id=(lax.rem(m + d, N_DEV),),
                device_id_type=pl.DeviceIdType.MESH,
            )
        pl.semaphore_wait(barrier_sem, N_DEV - 1)
        wq_dma.wait()
        wo_dma.wait()

        def reduce_and_ag(w_idx):
            for s_ in range(N_DEV):
                @pl.when(s_ != m)
                def _(w_idx=w_idx, s_=s_):
                    rd = pltpu.make_async_remote_copy(
                        src_ref=rs_ref.at[w_idx, s_],
                        dst_ref=rs_ref.at[w_idx, s_],
                        send_sem=s1_sems.at[0],
                        recv_sem=r1_sems.at[w_idx, s_],
                        device_id=(0,),
                        device_id_type=pl.DeviceIdType.MESH,
                    )
                    rd.wait_recv()
            total = (rs_ref[w_idx, 0].astype(jnp.float32)
                     + rs_ref[w_idx, 1].astype(jnp.float32)
                     + rs_ref[w_idx, 2].astype(jnp.float32)
                     + rs_ref[w_idx, 3].astype(jnp.float32))
            j_own = w_idx * N_DEV + m
            out_ref[0, pl.ds(j_own * CR, CR), :] = total
            ag_ref[j_own] = total.astype(COMM_DT)
            for d in range(1, N_DEV):
                tgt = lax.rem(m + d, N_DEV)
                rdma = pltpu.make_async_remote_copy(
                    src_ref=ag_ref.at[j_own],
                    dst_ref=ag_ref.at[j_own],
                    send_sem=s2_sems.at[w_idx, d - 1],
                    recv_sem=r2_sems.at[m, w_idx],
                    device_id=(tgt,),
                    device_id_type=pl.DeviceIdType.MESH,
                )
                rdma.start()


        for c in range(NQB):
            rows = slice(c * QB, (c + 1) * QB)
            kvlen = QB * (c + 1)
            q_blk = jnp.dot(x_ref[rows, :], wq_sl[...],
                            preferred_element_type=jnp.float32)

            row_blk = (lax.broadcasted_iota(jnp.int32, (QB, kvlen), 0)
                       + c * QB) // BLK
            col_blk = lax.broadcasted_iota(jnp.int32, (QB, kvlen), 1) // BLK
            mask = col_blk <= row_blk

            for h in range(H_PER):
                sl = slice(h * DH, (h + 1) * DH)
                s = lax.dot_general(
                    q_blk[:, sl], k_ref[:kvlen, sl],
                    (((1,), (1,)), ((), ())),
                    preferred_element_type=jnp.float32) * SCALE
                s = jnp.where(mask, s, -1e9)
                mx = jnp.max(s, axis=1, keepdims=True)
                w = jnp.exp(s - mx)
                p = w / jnp.sum(w, axis=1, keepdims=True)
                ctx_ref[rows, sl] = jnp.dot(
                    p, v_ref[:kvlen, sl], preferred_element_type=jnp.float32)

            for t in range(CPB):
                j = c * CPB + t
                w_idx = j // N_DEV
                o = j % N_DEV
                pr = jnp.dot(ctx_ref[j * CR:(j + 1) * CR, :], wo_sl[...],
                             preferred_element_type=jnp.float32)
                pr16 = pr.astype(COMM_DT)
                stage_ref[j] = pr16

                @pl.when(m == o)
                def _(w_idx=w_idx, pr16=pr16):
                    rs_ref[w_idx, m] = pr16

                @pl.when(m != o)
                def _(j=j, w_idx=w_idx, o=o):
                    rdma = pltpu.make_async_remote_copy(
                        src_ref=stage_ref.at[j],
                        dst_ref=rs_ref.at[w_idx, m],
                        send_sem=s1_sems.at[j],
                        recv_sem=r1_sems.at[w_idx, m],
                        device_id=(o,),
                        device_id_type=pl.DeviceIdType.MESH,
                    )
                    rdma.start()

            if c >= 1:
                reduce_and_ag(c - 1)

        reduce_and_ag(W_OWN - 1)

        for w_idx in range(W_OWN):
            for s_ in range(N_DEV):
                @pl.when(s_ != m)
                def _(s_=s_, w_idx=w_idx):
                    jj = w_idx * N_DEV + s_
                    rd = pltpu.make_async_remote_copy(
                        src_ref=ag_ref.at[jj],
                        dst_ref=ag_ref.at[jj],
                        send_sem=s2_sems.at[0, 0],
                        recv_sem=r2_sems.at[s_, w_idx],
                        device_id=(0,),
                        device_id_type=pl.DeviceIdType.MESH,
                    )
                    rd.wait_recv()
                    out_ref[0, jj * CR:(jj + 1) * CR, :] = (
                        ag_ref[jj].astype(jnp.float32))

        for j in range(NC):
            @pl.when(m != j % N_DEV)
            def _(j=j):
                rd = pltpu.make_async_remote_copy(
                    src_ref=stage_ref.at[j],
                    dst_ref=rs_ref.at[j // N_DEV, m],
                    send_sem=s1_sems.at[j],
                    recv_sem=r1_sems.at[0, 0],
                    device_id=(0,),
                    device_id_type=pl.DeviceIdType.MESH,
                )
                rd.wait_send()
        for w_idx in range(W_OWN):
            for d in range(1, N_DEV):
                rd = pltpu.make_async_remote_copy(
                    src_ref=ag_ref.at[0],
                    dst_ref=ag_ref.at[0],
                    send_sem=s2_sems.at[w_idx, d - 1],
                    recv_sem=r2_sems.at[0, 0],
                    device_id=(0,),
                    device_id_type=pl.DeviceIdType.MESH,
                )
                rd.wait_send()

    out = pl.pallas_call(
        body,
        out_shape=jax.ShapeDtypeStruct((1, SQ, D_MODEL), jnp.float32),
        in_specs=[
            pl.BlockSpec(memory_space=pltpu.VMEM),
            pl.BlockSpec(memory_space=pltpu.MemorySpace.HBM),
            pl.BlockSpec(memory_space=pltpu.VMEM),
            pl.BlockSpec(memory_space=pltpu.VMEM),
            pl.BlockSpec(memory_space=pltpu.MemorySpace.HBM),
        ],
        out_specs=pl.BlockSpec(memory_space=pltpu.VMEM),
        scratch_shapes=[
            pltpu.VMEM((D_MODEL, D_MODEL), jnp.float32),
            pltpu.VMEM((D_MODEL, D_MODEL), jnp.float32),
            pltpu.VMEM((SQ, H_PER * DH), jnp.float32),
            pltpu.VMEM((NC, CR, D_MODEL), COMM_DT),
            pltpu.VMEM((W_OWN, N_DEV, CR, D_MODEL), COMM_DT),
            pltpu.VMEM((NC, CR, D_MODEL), COMM_DT),
            pltpu.SemaphoreType.DMA((2,)),
            pltpu.SemaphoreType.DMA((NC,)),
            pltpu.SemaphoreType.DMA((W_OWN, N_DEV)),
            pltpu.SemaphoreType.DMA((W_OWN, N_DEV - 1)),
            pltpu.SemaphoreType.DMA((N_DEV, W_OWN)),
        ],
        compiler_params=pltpu.CompilerParams(collective_id=0),
    )(x2, Wq, K2, V2, Wo)

    return out
